# baseline (speedup 1.0000x reference)
import functools

import jax
import jax.numpy as jnp
from jax.experimental import pallas as pl
from jax.experimental.pallas import tpu as pltpu

_THRESHOLD = 0.5
_NEG_POS_RATIO = 3
_ALPHA = 1.0
_GAMMA = 2.0


def _pass1(boxes_ref, priors_ref, ov_ref, obj_ref, bp_ref, *, n_priors):
    boxes = boxes_ref[0]
    no = boxes.shape[0]
    pc = priors_ref[...]
    pcx, pcy, pw, ph = pc[0:1], pc[1:2], pc[2:3], pc[3:4]
    px1 = pcx - pw * 0.5
    py1 = pcy - ph * 0.5
    px2 = pcx + pw * 0.5
    py2 = pcy + ph * 0.5

    bx1, by1, bx2, by2 = boxes[:, 0:1], boxes[:, 1:2], boxes[:, 2:3], boxes[:, 3:4]

    iw = jnp.maximum(jnp.minimum(bx2, px2) - jnp.maximum(bx1, px1), 0.0)
    ih = jnp.maximum(jnp.minimum(by2, py2) - jnp.maximum(by1, py1), 0.0)
    inter = iw * ih
    area_b = (bx2 - bx1) * (by2 - by1)
    area_p = (px2 - px1) * (py2 - py1)
    ov = inter / (area_b + area_p - inter)

    i_obj = jax.lax.broadcasted_iota(jnp.int32, ov.shape, 0)
    lane = jax.lax.broadcasted_iota(jnp.int32, ov.shape, 1)

    ovmax = jnp.max(ov, axis=0, keepdims=True)
    obj = jnp.min(jnp.where(ov == ovmax, i_obj, no), axis=0, keepdims=True)
    ov_ref[0] = ovmax
    obj_ref[0] = obj

    tmax = jnp.max(ov, axis=1, keepdims=True)
    bp_ref[0] = jnp.min(jnp.where(ov == tmax, lane, n_priors), axis=1,
                        keepdims=True)


def _pass2(locs_ref, scores_ref, table_ref, priors_ref, ov_ref, obj_ref, bp_ref,
           cn_ref, stats_ref):
    ovmax = ov_ref[0]
    obj = obj_ref[0]
    bp = bp_ref[0]
    no = bp.shape[0]
    p = obj.shape[1]

    i_obj = jax.lax.broadcasted_iota(jnp.int32, (no, p), 0)
    lane = jax.lax.broadcasted_iota(jnp.int32, (no, p), 1)

    match = bp == lane
    forced = jnp.max(jnp.where(match, i_obj, -1), axis=0, keepdims=True)
    hasf = forced >= 0
    obj_e = jnp.where(hasf, forced, obj)
    ov_e = jnp.where(hasf, 1.0, ovmax)

    onehot = (i_obj == obj_e).astype(jnp.float32)
    table = table_ref[0]
    g = jax.lax.dot_general(
        table, onehot, (((1,), (0,)), ((), ())),
        precision=jax.lax.Precision.HIGHEST,
        preferred_element_type=jnp.float32)
    x1, y1, x2, y2 = g[0:1], g[1:2], g[2:3], g[3:4]
    lab = jnp.where(ov_e < _THRESHOLD, 0.0, g[4:5])
    posf = (lab > 0.5).astype(jnp.float32)

    pc = priors_ref[...]
    pcx, pcy, pw, ph = pc[0:1], pc[1:2], pc[2:3], pc[3:4]
    gx = (0.5 * (x1 + x2) - pcx) / (pw * 0.1)
    gy = (0.5 * (y1 + y2) - pcy) / (ph * 0.1)
    gw = jnp.log((x2 - x1) / pw) * 5.0
    gh = jnp.log((y2 - y1) / ph) * 5.0
    tl = jnp.concatenate([gx, gy, gw, gh], axis=0)

    d = locs_ref[0] - tl
    ad = jnp.abs(d)
    sl1 = jnp.where(ad < 1.0, 0.5 * d * d, ad - 0.5)
    loc_part = jnp.sum(sl1 * posf)

    x = scores_ref[0]
    m = jnp.max(x, axis=0, keepdims=True)
    xm = x - m
    lse = jnp.log(jnp.sum(jnp.exp(xm), axis=0, keepdims=True))
    labi = lab.astype(jnp.int32)
    i_cls = jax.lax.broadcasted_iota(jnp.int32, x.shape, 0)
    xt = jnp.sum(jnp.where(i_cls == labi, xm, 0.0), axis=0, keepdims=True)
    logpt = xt - lse
    pt = jnp.exp(logpt)
    om = 1.0 - pt
    conf = om * om * (-logpt)

    pos_part = jnp.sum(conf * posf)
    np_part = jnp.sum(posf)
    cn_ref[0] = conf * (1.0 - posf)
    stats_ref[0] = jnp.concatenate([
        jnp.full((1, 128), np_part, jnp.float32),
        jnp.full((1, 128), pos_part, jnp.float32),
        jnp.full((1, 128), loc_part, jnp.float32),
    ], axis=0)


def _mine_combine(cn_ref, stats_ref, out_ref, *, n_priors):
    v = cn_ref[...]
    npos = stats_ref[:, 0, 0:1]
    psum = stats_ref[:, 1, 0:1]
    lloc = stats_ref[:, 2, 0:1]
    keff = jnp.minimum(_NEG_POS_RATIO * npos, float(n_priors))

    def body(i, acc):
        bit = jax.lax.shift_left(jnp.int32(1), jnp.int32(30) - i)
        cand = acc | bit
        cand_f = jax.lax.bitcast_convert_type(cand, jnp.float32)
        cnt = jnp.sum((v >= cand_f).astype(jnp.float32), axis=1, keepdims=True)
        take = jnp.logical_and(cnt >= keff, keff > 0)
        return jnp.where(take, cand, acc)

    acc = jax.lax.fori_loop(0, 31, body, jnp.zeros(npos.shape, jnp.int32))
    tk = jax.lax.bitcast_convert_type(acc, jnp.float32)
    gt = v > tk
    cnt_gt = jnp.sum(gt.astype(jnp.float32), axis=1, keepdims=True)
    sum_gt = jnp.sum(jnp.where(gt, v, 0.0), axis=1, keepdims=True)
    hn = jnp.where(keff > 0, sum_gt + (keff - cnt_gt) * tk, 0.0)

    conf_loss = (jnp.sum(hn) + jnp.sum(psum)) / jnp.sum(npos)
    total = conf_loss + _ALPHA * jnp.sum(lloc)
    out_ref[...] = jnp.full((1, 1), total, dtype=jnp.float32)


def kernel(predicted_locs, predicted_scores, boxes, labels, priors_cxcy):
    b, p, _ = predicted_locs.shape
    c = predicted_scores.shape[2]
    no = boxes.shape[1]

    priors_t = priors_cxcy.T
    labels_f = labels.astype(jnp.float32)
    table = jnp.concatenate(
        [jnp.transpose(boxes, (0, 2, 1)), labels_f[:, None, :]], axis=1)
    h = b // 2
    locs_th = [jnp.transpose(predicted_locs[i * h:(i + 1) * h], (0, 2, 1))
               for i in range(2)]
    scores_th = [jnp.transpose(predicted_scores[i * h:(i + 1) * h], (0, 2, 1))
                 for i in range(2)]

    ov, obj, bp = pl.pallas_call(
        functools.partial(_pass1, n_priors=p),
        grid=(b,),
        in_specs=[
            pl.BlockSpec((1, no, 4), lambda i: (i, 0, 0)),
            pl.BlockSpec((4, p), lambda i: (0, 0)),
        ],
        out_specs=[
            pl.BlockSpec((1, 1, p), lambda i: (i, 0, 0)),
            pl.BlockSpec((1, 1, p), lambda i: (i, 0, 0)),
            pl.BlockSpec((1, no, 1), lambda i: (i, 0, 0)),
        ],
        out_shape=[
            jax.ShapeDtypeStruct((b, 1, p), jnp.float32),
            jax.ShapeDtypeStruct((b, 1, p), jnp.int32),
            jax.ShapeDtypeStruct((b, no, 1), jnp.int32),
        ],
        compiler_params=pltpu.CompilerParams(
            dimension_semantics=("parallel",)),
    )(boxes, priors_t)

    def run_pass2(locs_h, scores_h, table_h, ov_h, obj_h, bp_h):
        bh = locs_h.shape[0]
        return pl.pallas_call(
            _pass2,
            grid=(bh,),
            in_specs=[
                pl.BlockSpec((1, 4, p), lambda i: (i, 0, 0)),
                pl.BlockSpec((1, c, p), lambda i: (i, 0, 0)),
                pl.BlockSpec((1, 5, no), lambda i: (i, 0, 0)),
                pl.BlockSpec((4, p), lambda i: (0, 0)),
                pl.BlockSpec((1, 1, p), lambda i: (i, 0, 0)),
                pl.BlockSpec((1, 1, p), lambda i: (i, 0, 0)),
                pl.BlockSpec((1, no, 1), lambda i: (i, 0, 0)),
            ],
            out_specs=[
                pl.BlockSpec((1, 1, p), lambda i: (i, 0, 0)),
                pl.BlockSpec((1, 3, 128), lambda i: (i, 0, 0)),
            ],
            out_shape=[
                jax.ShapeDtypeStruct((bh, 1, p), jnp.float32),
                jax.ShapeDtypeStruct((bh, 3, 128), jnp.float32),
            ],
            compiler_params=pltpu.CompilerParams(
                dimension_semantics=("parallel",)),
        )(locs_h, scores_h, table_h, priors_t, ov_h, obj_h, bp_h)

    halves = [
        run_pass2(locs_th[i], scores_th[i], table[i * h:(i + 1) * h],
                  ov[i * h:(i + 1) * h], obj[i * h:(i + 1) * h],
                  bp[i * h:(i + 1) * h])
        for i in range(2)
    ]
    cn = jnp.concatenate([halves[0][0], halves[1][0]], axis=0)
    stats = jnp.concatenate([halves[0][1], halves[1][1]], axis=0)

    out = pl.pallas_call(
        functools.partial(_mine_combine, n_priors=p),
        in_specs=[
            pl.BlockSpec((b, p), lambda: (0, 0)),
            pl.BlockSpec((b, 3, 128), lambda: (0, 0, 0)),
        ],
        out_specs=pl.BlockSpec((1, 1), lambda: (0, 0)),
        out_shape=jax.ShapeDtypeStruct((1, 1), jnp.float32),
    )(cn.reshape(b, p), stats)

    return out[0, 0]


# speedup vs baseline: 1.3891x; 1.3891x over previous
import functools

import jax
import jax.numpy as jnp
from jax.experimental import pallas as pl
from jax.experimental.pallas import tpu as pltpu

_THRESHOLD = 0.5
_NEG_POS_RATIO = 3
_ALPHA = 1.0
_GAMMA = 2.0


def _pass1(boxes_ref, priors_ref, ov_ref, obj_ref, bp_ref, *, n_priors):
    boxes = boxes_ref[0]
    no = boxes.shape[0]
    pc = priors_ref[...]
    pcx, pcy, pw, ph = pc[0:1], pc[1:2], pc[2:3], pc[3:4]
    px1 = pcx - pw * 0.5
    py1 = pcy - ph * 0.5
    px2 = pcx + pw * 0.5
    py2 = pcy + ph * 0.5

    bx1, by1, bx2, by2 = boxes[:, 0:1], boxes[:, 1:2], boxes[:, 2:3], boxes[:, 3:4]

    iw = jnp.maximum(jnp.minimum(bx2, px2) - jnp.maximum(bx1, px1), 0.0)
    ih = jnp.maximum(jnp.minimum(by2, py2) - jnp.maximum(by1, py1), 0.0)
    inter = iw * ih
    area_b = (bx2 - bx1) * (by2 - by1)
    area_p = (px2 - px1) * (py2 - py1)
    ov = inter / (area_b + area_p - inter)

    i_obj = jax.lax.broadcasted_iota(jnp.int32, ov.shape, 0)
    lane = jax.lax.broadcasted_iota(jnp.int32, ov.shape, 1)

    ovmax = jnp.max(ov, axis=0, keepdims=True)
    obj = jnp.min(jnp.where(ov == ovmax, i_obj, no), axis=0, keepdims=True)
    ov_ref[0] = ovmax
    obj_ref[0] = obj

    tmax = jnp.max(ov, axis=1, keepdims=True)
    bp_ref[0] = jnp.min(jnp.where(ov == tmax, lane, n_priors), axis=1,
                        keepdims=True)


def _pass2(locs_ref, scores_ref, table_ref, priors_ref, ov_ref, obj_ref, bp_ref,
           cn_ref, stats_ref):
    ovmax = ov_ref[0]
    obj = obj_ref[0]
    bp = bp_ref[0]
    no = bp.shape[0]
    p = obj.shape[1]

    i_obj = jax.lax.broadcasted_iota(jnp.int32, (no, p), 0)
    lane = jax.lax.broadcasted_iota(jnp.int32, (no, p), 1)

    match = bp == lane
    forced = jnp.max(jnp.where(match, i_obj, -1), axis=0, keepdims=True)
    hasf = forced >= 0
    obj_e = jnp.where(hasf, forced, obj)
    ov_e = jnp.where(hasf, 1.0, ovmax)

    onehot = (i_obj == obj_e).astype(jnp.float32)
    table = table_ref[0]
    g = jax.lax.dot_general(
        table, onehot, (((1,), (0,)), ((), ())),
        precision=jax.lax.Precision.HIGHEST,
        preferred_element_type=jnp.float32)
    x1, y1, x2, y2 = g[0:1], g[1:2], g[2:3], g[3:4]
    lab = jnp.where(ov_e < _THRESHOLD, 0.0, g[4:5])
    posf = (lab > 0.5).astype(jnp.float32)

    pc = priors_ref[...]
    pcx, pcy, pw, ph = pc[0:1], pc[1:2], pc[2:3], pc[3:4]
    gx = (0.5 * (x1 + x2) - pcx) / (pw * 0.1)
    gy = (0.5 * (y1 + y2) - pcy) / (ph * 0.1)
    gw = jnp.log((x2 - x1) / pw) * 5.0
    gh = jnp.log((y2 - y1) / ph) * 5.0
    tl = jnp.concatenate([gx, gy, gw, gh], axis=0)

    d = locs_ref[0] - tl
    ad = jnp.abs(d)
    sl1 = jnp.where(ad < 1.0, 0.5 * d * d, ad - 0.5)
    loc_part = jnp.sum(sl1 * posf)

    x = scores_ref[0]
    m = jnp.max(x, axis=0, keepdims=True)
    xm = x - m
    lse = jnp.log(jnp.sum(jnp.exp(xm), axis=0, keepdims=True))
    labi = lab.astype(jnp.int32)
    i_cls = jax.lax.broadcasted_iota(jnp.int32, x.shape, 0)
    xt = jnp.sum(jnp.where(i_cls == labi, xm, 0.0), axis=0, keepdims=True)
    logpt = xt - lse
    pt = jnp.exp(logpt)
    om = 1.0 - pt
    conf = om * om * (-logpt)

    pos_part = jnp.sum(conf * posf)
    np_part = jnp.sum(posf)
    cn_ref[0] = conf * (1.0 - posf)
    stats_ref[0] = jnp.concatenate([
        jnp.full((1, 128), np_part, jnp.float32),
        jnp.full((1, 128), pos_part, jnp.float32),
        jnp.full((1, 128), loc_part, jnp.float32),
    ], axis=0)


def _mine_combine(cn_ref, stats_ref, out_ref, *, n_priors):
    v = cn_ref[...]
    npos = stats_ref[:, 0, 0:1]
    psum = stats_ref[:, 1, 0:1]
    lloc = stats_ref[:, 2, 0:1]
    keff = jnp.minimum(_NEG_POS_RATIO * npos, float(n_priors))

    def body(i, acc):
        bit = jax.lax.shift_left(jnp.int32(1), jnp.int32(30) - i)
        cand = acc | bit
        cand_f = jax.lax.bitcast_convert_type(cand, jnp.float32)
        cnt = jnp.sum((v >= cand_f).astype(jnp.float32), axis=1, keepdims=True)
        take = jnp.logical_and(cnt >= keff, keff > 0)
        return jnp.where(take, cand, acc)

    acc = jax.lax.fori_loop(0, 31, body, jnp.zeros(npos.shape, jnp.int32))
    tk = jax.lax.bitcast_convert_type(acc, jnp.float32)
    gt = v > tk
    cnt_gt = jnp.sum(gt.astype(jnp.float32), axis=1, keepdims=True)
    sum_gt = jnp.sum(jnp.where(gt, v, 0.0), axis=1, keepdims=True)
    hn = jnp.where(keff > 0, sum_gt + (keff - cnt_gt) * tk, 0.0)

    conf_loss = (jnp.sum(hn) + jnp.sum(psum)) / jnp.sum(npos)
    total = conf_loss + _ALPHA * jnp.sum(lloc)
    out_ref[...] = jnp.full((1, 1), total, dtype=jnp.float32)


def kernel(predicted_locs, predicted_scores, boxes, labels, priors_cxcy):
    b, p, _ = predicted_locs.shape
    c = predicted_scores.shape[2]
    no = boxes.shape[1]

    locs_t = jnp.transpose(predicted_locs, (0, 2, 1))
    scores_t = jnp.transpose(predicted_scores, (0, 2, 1))
    priors_t = priors_cxcy.T
    labels_f = labels.astype(jnp.float32)
    table = jnp.concatenate(
        [jnp.transpose(boxes, (0, 2, 1)), labels_f[:, None, :]], axis=1)

    ov, obj, bp = pl.pallas_call(
        functools.partial(_pass1, n_priors=p),
        grid=(b,),
        in_specs=[
            pl.BlockSpec((1, no, 4), lambda i: (i, 0, 0)),
            pl.BlockSpec((4, p), lambda i: (0, 0)),
        ],
        out_specs=[
            pl.BlockSpec((1, 1, p), lambda i: (i, 0, 0)),
            pl.BlockSpec((1, 1, p), lambda i: (i, 0, 0)),
            pl.BlockSpec((1, no, 1), lambda i: (i, 0, 0)),
        ],
        out_shape=[
            jax.ShapeDtypeStruct((b, 1, p), jnp.float32),
            jax.ShapeDtypeStruct((b, 1, p), jnp.int32),
            jax.ShapeDtypeStruct((b, no, 1), jnp.int32),
        ],
        compiler_params=pltpu.CompilerParams(
            dimension_semantics=("parallel",)),
    )(boxes, priors_t)

    cn, stats = pl.pallas_call(
        _pass2,
        grid=(b,),
        in_specs=[
            pl.BlockSpec((1, 4, p), lambda i: (i, 0, 0)),
            pl.BlockSpec((1, c, p), lambda i: (i, 0, 0)),
            pl.BlockSpec((1, 5, no), lambda i: (i, 0, 0)),
            pl.BlockSpec((4, p), lambda i: (0, 0)),
            pl.BlockSpec((1, 1, p), lambda i: (i, 0, 0)),
            pl.BlockSpec((1, 1, p), lambda i: (i, 0, 0)),
            pl.BlockSpec((1, no, 1), lambda i: (i, 0, 0)),
        ],
        out_specs=[
            pl.BlockSpec((1, 1, p), lambda i: (i, 0, 0)),
            pl.BlockSpec((1, 3, 128), lambda i: (i, 0, 0)),
        ],
        out_shape=[
            jax.ShapeDtypeStruct((b, 1, p), jnp.float32),
            jax.ShapeDtypeStruct((b, 3, 128), jnp.float32),
        ],
        compiler_params=pltpu.CompilerParams(
            dimension_semantics=("parallel",)),
    )(locs_t, scores_t, table, priors_t, ov, obj, bp)

    out = pl.pallas_call(
        functools.partial(_mine_combine, n_priors=p),
        in_specs=[
            pl.BlockSpec((b, p), lambda: (0, 0)),
            pl.BlockSpec((b, 3, 128), lambda: (0, 0, 0)),
        ],
        out_specs=pl.BlockSpec((1, 1), lambda: (0, 0)),
        out_shape=jax.ShapeDtypeStruct((1, 1), jnp.float32),
    )(cn.reshape(b, p), stats)

    return out[0, 0]


# speedup vs baseline: 1.4260x; 1.0265x over previous
import functools

import jax
import jax.numpy as jnp
from jax.experimental import pallas as pl
from jax.experimental.pallas import tpu as pltpu

_THRESHOLD = 0.5
_NEG_POS_RATIO = 3
_ALPHA = 1.0
_GAMMA = 2.0


def _pass1(boxes_ref, priors_ref, ov_ref, obj_ref, bp_ref, *, n_priors):
    boxes = boxes_ref[0]
    no = boxes.shape[0]
    pc = priors_ref[...]
    pcx, pcy, pw, ph = pc[0:1], pc[1:2], pc[2:3], pc[3:4]
    px1 = pcx - pw * 0.5
    py1 = pcy - ph * 0.5
    px2 = pcx + pw * 0.5
    py2 = pcy + ph * 0.5

    bx1, by1, bx2, by2 = boxes[:, 0:1], boxes[:, 1:2], boxes[:, 2:3], boxes[:, 3:4]

    iw = jnp.maximum(jnp.minimum(bx2, px2) - jnp.maximum(bx1, px1), 0.0)
    ih = jnp.maximum(jnp.minimum(by2, py2) - jnp.maximum(by1, py1), 0.0)
    inter = iw * ih
    area_b = (bx2 - bx1) * (by2 - by1)
    area_p = (px2 - px1) * (py2 - py1)
    ov = inter / (area_b + area_p - inter)

    i_obj = jax.lax.broadcasted_iota(jnp.int32, ov.shape, 0)
    lane = jax.lax.broadcasted_iota(jnp.int32, ov.shape, 1)

    ovmax = jnp.max(ov, axis=0, keepdims=True)
    obj = jnp.min(jnp.where(ov == ovmax, i_obj, no), axis=0, keepdims=True)
    ov_ref[0] = ovmax
    obj_ref[0] = obj

    tmax = jnp.max(ov, axis=1, keepdims=True)
    bp_ref[0] = jnp.min(jnp.where(ov == tmax, lane, n_priors), axis=1,
                        keepdims=True)


def _pass2(locs_ref, scores_ref, table_ref, priors_ref, ov_ref, obj_ref, bp_ref,
           out_ref, cn_s, stats_s, *, n_images, n_priors):
    ib = pl.program_id(0)
    ovmax = ov_ref[0]
    obj = obj_ref[0]
    bp = bp_ref[0]
    no = bp.shape[0]
    p = obj.shape[1]

    i_obj = jax.lax.broadcasted_iota(jnp.int32, (no, p), 0)
    lane = jax.lax.broadcasted_iota(jnp.int32, (no, p), 1)

    match = bp == lane
    forced = jnp.max(jnp.where(match, i_obj, -1), axis=0, keepdims=True)
    hasf = forced >= 0
    obj_e = jnp.where(hasf, forced, obj)
    ov_e = jnp.where(hasf, 1.0, ovmax)

    onehot = (i_obj == obj_e).astype(jnp.float32)
    table = table_ref[0]
    g = jax.lax.dot_general(
        table, onehot, (((1,), (0,)), ((), ())),
        precision=jax.lax.Precision.HIGHEST,
        preferred_element_type=jnp.float32)
    x1, y1, x2, y2 = g[0:1], g[1:2], g[2:3], g[3:4]
    lab = jnp.where(ov_e < _THRESHOLD, 0.0, g[4:5])
    posf = (lab > 0.5).astype(jnp.float32)

    pc = priors_ref[...]
    pcx, pcy, pw, ph = pc[0:1], pc[1:2], pc[2:3], pc[3:4]
    gx = (0.5 * (x1 + x2) - pcx) / (pw * 0.1)
    gy = (0.5 * (y1 + y2) - pcy) / (ph * 0.1)
    gw = jnp.log((x2 - x1) / pw) * 5.0
    gh = jnp.log((y2 - y1) / ph) * 5.0
    tl = jnp.concatenate([gx, gy, gw, gh], axis=0)

    d = locs_ref[0] - tl
    ad = jnp.abs(d)
    sl1 = jnp.where(ad < 1.0, 0.5 * d * d, ad - 0.5)
    loc_part = jnp.sum(sl1 * posf)

    x = scores_ref[0]
    m = jnp.max(x, axis=0, keepdims=True)
    xm = x - m
    lse = jnp.log(jnp.sum(jnp.exp(xm), axis=0, keepdims=True))
    labi = lab.astype(jnp.int32)
    i_cls = jax.lax.broadcasted_iota(jnp.int32, x.shape, 0)
    xt = jnp.sum(jnp.where(i_cls == labi, xm, 0.0), axis=0, keepdims=True)
    logpt = xt - lse
    pt = jnp.exp(logpt)
    om = 1.0 - pt
    conf = om * om * (-logpt)

    pos_part = jnp.sum(conf * posf)
    np_part = jnp.sum(posf)
    cn_s[pl.ds(ib, 1), :] = conf * (1.0 - posf)
    stats_s[pl.ds(ib, 1), :] = jnp.concatenate([
        jnp.full((1, 1), np_part, jnp.float32),
        jnp.full((1, 1), pos_part, jnp.float32),
        jnp.full((1, 1), loc_part, jnp.float32),
    ], axis=1)

    @pl.when(ib == n_images - 1)
    def _():
        _mine_combine_body(cn_s[...], stats_s[...], out_ref, n_priors)


def _mine_combine_body(v, stats, out_ref, n_priors):
    npos = stats[:, 0:1]
    psum = stats[:, 1:2]
    lloc = stats[:, 2:3]
    keff = jnp.minimum(_NEG_POS_RATIO * npos, float(n_priors))

    def body(i, acc):
        bit = jax.lax.shift_left(jnp.int32(1), jnp.int32(30) - i)
        cand = acc | bit
        cand_f = jax.lax.bitcast_convert_type(cand, jnp.float32)
        cnt = jnp.sum((v >= cand_f).astype(jnp.float32), axis=1, keepdims=True)
        take = jnp.logical_and(cnt >= keff, keff > 0)
        return jnp.where(take, cand, acc)

    acc = jax.lax.fori_loop(0, 31, body, jnp.zeros(npos.shape, jnp.int32))
    tk = jax.lax.bitcast_convert_type(acc, jnp.float32)
    gt = v > tk
    cnt_gt = jnp.sum(gt.astype(jnp.float32), axis=1, keepdims=True)
    sum_gt = jnp.sum(jnp.where(gt, v, 0.0), axis=1, keepdims=True)
    hn = jnp.where(keff > 0, sum_gt + (keff - cnt_gt) * tk, 0.0)

    conf_loss = (jnp.sum(hn) + jnp.sum(psum)) / jnp.sum(npos)
    total = conf_loss + _ALPHA * jnp.sum(lloc)
    out_ref[...] = jnp.full((1, 1), total, dtype=jnp.float32)


def kernel(predicted_locs, predicted_scores, boxes, labels, priors_cxcy):
    b, p, _ = predicted_locs.shape
    c = predicted_scores.shape[2]
    no = boxes.shape[1]

    locs_t = jnp.transpose(predicted_locs, (0, 2, 1))
    scores_t = jnp.transpose(predicted_scores, (0, 2, 1))
    priors_t = priors_cxcy.T
    labels_f = labels.astype(jnp.float32)
    table = jnp.concatenate(
        [jnp.transpose(boxes, (0, 2, 1)), labels_f[:, None, :]], axis=1)

    ov, obj, bp = pl.pallas_call(
        functools.partial(_pass1, n_priors=p),
        grid=(b,),
        in_specs=[
            pl.BlockSpec((1, no, 4), lambda i: (i, 0, 0)),
            pl.BlockSpec((4, p), lambda i: (0, 0)),
        ],
        out_specs=[
            pl.BlockSpec((1, 1, p), lambda i: (i, 0, 0)),
            pl.BlockSpec((1, 1, p), lambda i: (i, 0, 0)),
            pl.BlockSpec((1, no, 1), lambda i: (i, 0, 0)),
        ],
        out_shape=[
            jax.ShapeDtypeStruct((b, 1, p), jnp.float32),
            jax.ShapeDtypeStruct((b, 1, p), jnp.int32),
            jax.ShapeDtypeStruct((b, no, 1), jnp.int32),
        ],
        compiler_params=pltpu.CompilerParams(
            dimension_semantics=("parallel",)),
    )(boxes, priors_t)

    out = pl.pallas_call(
        functools.partial(_pass2, n_images=b, n_priors=p),
        grid=(b,),
        in_specs=[
            pl.BlockSpec((1, 4, p), lambda i: (i, 0, 0)),
            pl.BlockSpec((1, c, p), lambda i: (i, 0, 0)),
            pl.BlockSpec((1, 5, no), lambda i: (i, 0, 0)),
            pl.BlockSpec((4, p), lambda i: (0, 0)),
            pl.BlockSpec((1, 1, p), lambda i: (i, 0, 0)),
            pl.BlockSpec((1, 1, p), lambda i: (i, 0, 0)),
            pl.BlockSpec((1, no, 1), lambda i: (i, 0, 0)),
        ],
        out_specs=pl.BlockSpec((1, 1), lambda i: (0, 0)),
        out_shape=jax.ShapeDtypeStruct((1, 1), jnp.float32),
        scratch_shapes=[
            pltpu.VMEM((b, p), jnp.float32),
            pltpu.VMEM((b, 3), jnp.float32),
        ],
        compiler_params=pltpu.CompilerParams(
            dimension_semantics=("arbitrary",)),
    )(locs_t, scores_t, table, priors_t, ov, obj, bp)

    return out[0, 0]
